# baseline (speedup 1.0000x reference)
import functools
import math

import jax
import jax.numpy as jnp
from jax import lax
from jax.experimental import pallas as pl
from jax.experimental.pallas import tpu as pltpu
from jax.experimental.pallas import tpu_sc as plsc

D_MODEL = 64
SCALE = math.sqrt(D_MODEL)
NBUF = 4
SPLITS = ((0, 104), (104, 96))


@functools.cache
def _make_kernel(BATCH, SEQ, D):
    info = plsc.get_sparse_core_info()
    nc, ns = info.num_cores, info.num_subcores
    nw = nc * ns
    assert BATCH % (nw * NBUF) == 0
    b_per_w = BATCH // nw
    rows_per_w = b_per_w * SEQ
    mesh = plsc.VectorSubcoreMesh(core_axis_name="c", subcore_axis_name="s")

    @functools.partial(
        pl.kernel,
        mesh=mesh,
        out_type=jax.ShapeDtypeStruct((BATCH, SEQ, D), jnp.float32),
        scratch_types=[
            pltpu.VMEM((rows_per_w,), jnp.int32),
            pltpu.VMEM((NBUF, SEQ, D), jnp.float32),
        ]
        + [pltpu.SemaphoreType.DMA] * (2 * NBUF),
        compiler_params=pltpu.CompilerParams(use_tc_tiling_on_sc=False),
    )
    def emb_kernel(idx_hbm, table_hbm, out_hbm, idx_v, rows_v, *sems):
        gsem = sems[:NBUF]
        ssem = sems[NBUF:]
        wid = lax.axis_index("s") * nc + lax.axis_index("c")
        base = wid * rows_per_w
        base_b = wid * b_per_w

        pltpu.sync_copy(idx_hbm.at[pl.ds(base, rows_per_w)], idx_v)

        def fire_gather(cb, buf):
            for off, ln in SPLITS:
                idx_slice = idx_v.at[pl.ds(cb * SEQ + off, ln)]
                pltpu.async_copy(
                    table_hbm.at[idx_slice],
                    rows_v.at[buf, pl.ds(off, ln)],
                    gsem[buf],
                )

        def wait_gather(buf):
            pltpu.make_async_copy(
                table_hbm.at[idx_v.at[pl.ds(0, SEQ)]], rows_v.at[buf], gsem[buf]
            ).wait()

        def fire_scatter(cb, buf):
            pltpu.async_copy(rows_v.at[buf], out_hbm.at[base_b + cb], ssem[buf])

        def wait_scatter(buf):
            pltpu.make_async_copy(
                rows_v.at[buf], out_hbm.at[base_b], ssem[buf]
            ).wait()

        fire_gather(0, 0)
        fire_gather(1, 1)

        def group_body(p, carry):
            for b in range(NBUF):
                cb = NBUF * p + b
                wait_gather(b)
                fire_scatter(cb, b)
                nb = (b + 2) % NBUF

                @pl.when(cb + 2 < b_per_w)
                def _():
                    @pl.when(cb >= 2)
                    def _():
                        wait_scatter(nb)

                    fire_gather(cb + 2, nb)

            return carry

        lax.fori_loop(0, b_per_w // NBUF, group_body, 0)
        for b in range(NBUF):
            wait_scatter(b)

    return emb_kernel


@jax.jit
def kernel(x, table):
    idx = x.reshape(-1).astype(jnp.int32)
    raw = _make_kernel(x.shape[0], x.shape[1], table.shape[1])(idx, table)
    return raw * jnp.float32(SCALE)


# speedup vs baseline: 1.2087x; 1.2087x over previous
import functools
import math

import jax
import jax.numpy as jnp
from jax import lax
from jax.experimental import pallas as pl
from jax.experimental.pallas import tpu as pltpu
from jax.experimental.pallas import tpu_sc as plsc

D_MODEL = 64
SCALE = math.sqrt(D_MODEL)
NBUF = 4
SPLITS = ((0, 104), (104, 96))


@functools.cache
def _make_kernel(BATCH, SEQ, D):
    info = plsc.get_sparse_core_info()
    nc, ns = info.num_cores, info.num_subcores
    nw = nc * ns
    assert BATCH % (nw * NBUF) == 0
    b_per_w = BATCH // nw
    rows_per_w = b_per_w * SEQ
    mesh = plsc.VectorSubcoreMesh(core_axis_name="c", subcore_axis_name="s")

    @functools.partial(
        pl.kernel,
        mesh=mesh,
        out_type=jax.ShapeDtypeStruct((BATCH, SEQ, D), jnp.float32),
        scratch_types=[
            pltpu.VMEM((rows_per_w,), jnp.int32),
            pltpu.VMEM((NBUF, SEQ, D), jnp.float32),
        ]
        + [pltpu.SemaphoreType.DMA] * (2 * NBUF),
        compiler_params=pltpu.CompilerParams(use_tc_tiling_on_sc=False),
    )
    def emb_kernel(idx_hbm, table_hbm, out_hbm, idx_v, rows_v, *sems):
        gsem = sems[:NBUF]
        ssem = sems[NBUF:]
        wid = lax.axis_index("s") * nc + lax.axis_index("c")
        base = wid * rows_per_w
        base_b = wid * b_per_w

        pltpu.sync_copy(idx_hbm.at[pl.ds(base, rows_per_w)], idx_v)

        def fire_gather(cb, buf):
            for off, ln in SPLITS:
                idx_slice = idx_v.at[pl.ds(cb * SEQ + off, ln)]
                pltpu.async_copy(
                    table_hbm.at[idx_slice],
                    rows_v.at[buf, pl.ds(off, ln)],
                    gsem[buf],
                )

        def wait_gather(buf):
            pltpu.make_async_copy(
                table_hbm.at[idx_v.at[pl.ds(0, SEQ)]], rows_v.at[buf], gsem[buf]
            ).wait()

        def fire_scatter(cb, buf):
            pltpu.async_copy(rows_v.at[buf], out_hbm.at[base_b + cb], ssem[buf])

        def wait_scatter(buf):
            pltpu.make_async_copy(
                rows_v.at[buf], out_hbm.at[base_b], ssem[buf]
            ).wait()

        fire_gather(0, 0)
        fire_gather(1, 1)

        def group_body(p, carry):
            for b in range(NBUF):
                cb = NBUF * p + b
                wait_gather(b)

                @plsc.parallel_loop(0, SEQ, unroll=8)
                def _(i):
                    for k in range(D // 16):
                        sl = pl.ds(k * 16, 16)
                        rows_v[b, i, sl] = rows_v[b, i, sl] * SCALE

                fire_scatter(cb, b)
                nb = (b + 2) % NBUF

                @pl.when(cb + 2 < b_per_w)
                def _():
                    @pl.when(cb >= 2)
                    def _():
                        wait_scatter(nb)

                    fire_gather(cb + 2, nb)

            return carry

        lax.fori_loop(0, b_per_w // NBUF, group_body, 0)
        for b in range(NBUF):
            wait_scatter(b)

    return emb_kernel


@jax.jit
def kernel(x, table):
    idx = x.reshape(-1).astype(jnp.int32)
    return _make_kernel(x.shape[0], x.shape[1], table.shape[1])(idx, table)
